# baseline (speedup 1.0000x reference)
import jax, jax.numpy as jnp
import numpy as np
from jax.experimental import pallas as pl

X_MIN, Y_MIN, Z_MIN, X_MAX, Y_MAX, Z_MAX = -40.0, -40.0, -3.0, 40.0, 40.0, 1.0
PX, PY = 0.16, 0.16
NX = int(np.round((X_MAX - X_MIN) / PX))
NY = int(np.round((Y_MAX - Y_MIN) / PY))
MAX_PTS = 100
MAX_PIL = 12000
OUT_CH = 64
EPS = 1e-5


def _precompute(p):
    n = p.shape[0]
    x, y, z = p[:, 0], p[:, 1], p[:, 2]
    m = (x >= X_MIN) & (x < X_MAX) & (y >= Y_MIN) & (y < Y_MAX) & (z >= Z_MIN) & (z < Z_MAX)
    xi = jnp.floor((x - X_MIN) / PX).astype(jnp.int32)
    yi = jnp.floor((y - Y_MIN) / PY).astype(jnp.int32)
    sent = NX * NY
    key = jnp.where(m, xi * NY + yi, sent)
    sk = jnp.sort(key)
    new = jnp.concatenate([jnp.ones((1,), dtype=bool), sk[1:] != sk[:-1]])
    uid = jnp.cumsum(new) - 1
    uk_arr = jnp.full((n,), sent, dtype=sk.dtype).at[uid].set(sk)
    P = jnp.sum(new & (sk < sent))
    P_eff = jnp.minimum(P, MAX_PIL)
    inv = jnp.searchsorted(uk_arr, key)
    v2 = m & (inv < MAX_PIL)
    seg = jnp.where(v2, inv, MAX_PIL)
    o1 = jnp.argsort(-z)
    order = o1[jnp.argsort(seg[o1])]
    seg_s = seg[order]
    counts = jnp.bincount(seg_s, length=MAX_PIL + 1)
    starts = jnp.concatenate([jnp.zeros((1,), counts.dtype), jnp.cumsum(counts)[:-1]])
    rank = jnp.arange(n, dtype=counts.dtype) - starts[seg_s]
    uk = uk_arr[:MAX_PIL]
    ux = uk // NY
    uy = uk % NY
    pm = jnp.arange(MAX_PIL) < P_eff
    xc = (ux * PX + X_MIN + PX / 2).astype(jnp.float32)
    yc = (uy * PY + Y_MIN + PY / 2).astype(jnp.float32)
    zcol = jnp.zeros((MAX_PIL,), dtype=ux.dtype)
    coords = jnp.stack([zcol, jnp.where(pm, ux, 0), jnp.where(pm, uy, 0)], axis=1).astype(jnp.int64)
    return dict(src=order, row=seg_s, col=rank, P_eff=P_eff, pm=pm, xc=xc, yc=yc, coords=coords)


def _bn(x, g, b, m, cnt):
    mean = (x * m).sum(axis=(0, 2), keepdims=True) / cnt
    var = (((x - mean) ** 2) * m).sum(axis=(0, 2), keepdims=True) / cnt
    return (x - mean) / jnp.sqrt(var + EPS) * g[None, :, None] + b[None, :, None]


def _pillar_feats(pts_b, pre, W1, g1, b1, W2, g2, b2):
    gathered = jnp.take(pts_b, pre['src'], axis=0)
    dense = jnp.zeros((MAX_PIL, MAX_PTS, 4), dtype=jnp.float32).at[pre['row'], pre['col']].set(gathered, mode='drop')
    xc = pre['xc'][:, None]
    yc = pre['yc'][:, None]
    enh = jnp.stack([
        jnp.broadcast_to(xc, (MAX_PIL, MAX_PTS)),
        jnp.broadcast_to(yc, (MAX_PIL, MAX_PTS)),
        dense[:, :, 0] - xc,
        dense[:, :, 1] - yc,
    ], axis=-1)
    feat = jnp.concatenate([dense, enh], axis=-1)
    pm = pre['pm']
    mask = pm[:, None, None].astype(jnp.float32)
    cnt = jnp.maximum(pre['P_eff'] * MAX_PTS, 1).astype(jnp.float32)
    h = jnp.einsum('pmc,oc->pom', feat, W1)
    h = jax.nn.relu(_bn(h, g1, b1, mask, cnt))
    h = jnp.einsum('pom,qo->pqm', h, W2)
    h = jax.nn.relu(_bn(h, g2, b2, mask, cnt))
    pooled = h.max(axis=-1)
    return jnp.where(pm[:, None], pooled, jnp.zeros((), dtype=jnp.float32))


def kernel(points, W1, g1, b1, W2, g2, b2):
    feats, coords = [], []
    for bi in range(points.shape[0]):
        pre = _precompute(points[bi])
        feats.append(_pillar_feats(points[bi], pre, W1, g1, b1, W2, g2, b2))
        coords.append(pre['coords'])
    return jnp.stack(feats, axis=0), jnp.stack(coords, axis=0)


# speedup vs baseline: 4.1209x; 4.1209x over previous
import jax
import jax.numpy as jnp
import numpy as np
from jax import lax
from jax.experimental import pallas as pl
from jax.experimental.pallas import tpu as pltpu

X_MIN, Y_MIN, Z_MIN, X_MAX, Y_MAX, Z_MAX = -40.0, -40.0, -3.0, 40.0, 40.0, 1.0
PX, PY = 0.16, 0.16
NX = int(np.round((X_MAX - X_MIN) / PX))
NY = int(np.round((Y_MAX - Y_MIN) / PY))
SENT = NX * NY
MAX_PTS = 100
MAX_PIL = 12000
OUT_CH = 64
EPS = 1e-5

TILE = 96
NTILES = MAX_PIL // TILE


def _prep(p):
    n = p.shape[0]
    x, y, z, w = p[:, 0], p[:, 1], p[:, 2], p[:, 3]
    m = ((x >= X_MIN) & (x < X_MAX) & (y >= Y_MIN) & (y < Y_MAX)
         & (z >= Z_MIN) & (z < Z_MAX))
    xi = jnp.floor((x - X_MIN) / PX).astype(jnp.int32)
    yi = jnp.floor((y - Y_MIN) / PY).astype(jnp.int32)
    key = jnp.where(m, xi * NY + yi, SENT).astype(jnp.int32)
    negz = -z
    sk, snz, sx, sy, sw = lax.sort((key, negz, x, y, w), num_keys=2,
                                   is_stable=True)
    pts_s = jnp.stack([sx, sy, -snz, sw], axis=1)

    valid = sk < SENT
    newg = jnp.concatenate([jnp.ones((1,), bool), sk[1:] != sk[:-1]])
    new_valid = newg & valid
    iota = jnp.arange(n, dtype=jnp.int32)
    startkey = jnp.where(new_valid, iota, n).astype(jnp.int32)
    s_sorted, uk_sorted = lax.sort((startkey, sk), num_keys=1)

    n_valid = jnp.sum(valid.astype(jnp.int32))
    P = jnp.sum(new_valid.astype(jnp.int32))
    P_eff = jnp.minimum(P, MAX_PIL)
    s_clip = jnp.minimum(s_sorted, n_valid)
    starts = s_clip[:MAX_PIL]
    counts = s_clip[1:MAX_PIL + 1] - starts
    c100 = jnp.minimum(counts, MAX_PTS)

    pts_pad = jnp.concatenate(
        [pts_s, jnp.zeros((MAX_PTS, 4), jnp.float32)], axis=0)
    dense = lax.gather(
        pts_pad, starts[:, None],
        lax.GatherDimensionNumbers(offset_dims=(1, 2),
                                   collapsed_slice_dims=(),
                                   start_index_map=(0,)),
        slice_sizes=(MAX_PTS, 4),
        mode=lax.GatherScatterMode.CLIP)
    cm = jnp.arange(MAX_PTS, dtype=jnp.int32)[None, :] < c100[:, None]
    dense = jnp.where(cm[:, :, None], dense, 0.0)

    pm = jnp.arange(MAX_PIL, dtype=jnp.int32) < P_eff
    uk = jnp.where(pm, uk_sorted[:MAX_PIL], 0)
    ux = uk // NY
    uy = uk % NY
    xc = (ux * PX + X_MIN + PX / 2).astype(jnp.float32)
    yc = (uy * PY + Y_MIN + PY / 2).astype(jnp.float32)
    zcol = jnp.zeros((MAX_PIL,), dtype=ux.dtype)
    coords = jnp.stack([zcol, ux, uy], axis=1).astype(jnp.int64)
    cnt = jnp.maximum(P_eff * MAX_PTS, 1).astype(jnp.float32)
    return (dense.reshape(MAX_PIL * MAX_PTS, 4), pm.astype(jnp.float32),
            xc, yc, coords, cnt)


def _mlp_body(cnt_ref, dense_ref, pm_ref, xc_ref, yc_ref, wm_ref, w2t_ref,
              gb_ref, out_ref, s1, s1q, s2, s2q, ab1, ab2):
    b = pl.program_id(0)
    ph = pl.program_id(1)
    t = pl.program_id(2)
    cnt = cnt_ref[b]

    wm = wm_ref[...]
    dense = dense_ref[0]
    h1 = jnp.dot(dense, wm[0:4, :], preferred_element_type=jnp.float32)
    h1 = h1.reshape(TILE, MAX_PTS, OUT_CH)
    xc = xc_ref[0]
    yc = yc_ref[0]
    bias = xc * wm[4:5, :] + yc * wm[5:6, :]
    h1 = h1 + bias[:, None, :]
    pm = pm_ref[0]
    pm3 = pm[:, :, None]

    @pl.when(ph == 0)
    def _phase0():
        @pl.when(t == 0)
        def _z0():
            s1[...] = jnp.zeros_like(s1)
            s1q[...] = jnp.zeros_like(s1q)
        h1m = h1 * pm3
        s1[...] += jnp.sum(h1m, axis=(0, 1))[None, :]
        s1q[...] += jnp.sum(h1m * h1m, axis=(0, 1))[None, :]
        out_ref[0] = jnp.zeros((TILE, OUT_CH), jnp.float32)

    @pl.when(ph == 1)
    def _phase1():
        @pl.when(t == 0)
        def _ab1():
            mean = s1[...] / cnt
            var = s1q[...] / cnt - mean * mean
            al = gb_ref[0:1, :] * lax.rsqrt(var + EPS)
            ab1[0:1, :] = al
            ab1[1:2, :] = gb_ref[1:2, :] - mean * al
            s2[...] = jnp.zeros_like(s2)
            s2q[...] = jnp.zeros_like(s2q)
        a1 = jnp.maximum(h1 * ab1[0:1, :][None] + ab1[1:2, :][None], 0.0)
        a1m = (a1 * pm3).reshape(TILE * MAX_PTS, OUT_CH)
        h2m = jnp.dot(a1m, w2t_ref[...], preferred_element_type=jnp.float32)
        s2[...] += jnp.sum(h2m, axis=0)[None, :]
        s2q[...] += jnp.sum(h2m * h2m, axis=0)[None, :]
        out_ref[0] = jnp.zeros((TILE, OUT_CH), jnp.float32)

    @pl.when(ph == 2)
    def _phase2():
        @pl.when(t == 0)
        def _ab2():
            mean = s2[...] / cnt
            var = s2q[...] / cnt - mean * mean
            al = gb_ref[2:3, :] * lax.rsqrt(var + EPS)
            ab2[0:1, :] = al
            ab2[1:2, :] = gb_ref[3:4, :] - mean * al
        a1 = jnp.maximum(h1 * ab1[0:1, :][None] + ab1[1:2, :][None], 0.0)
        h2 = jnp.dot(a1.reshape(TILE * MAX_PTS, OUT_CH), w2t_ref[...],
                     preferred_element_type=jnp.float32)
        a2 = jnp.maximum(h2 * ab2[0:1, :] + ab2[1:2, :], 0.0)
        pooled = jnp.max(a2.reshape(TILE, MAX_PTS, OUT_CH), axis=1)
        out_ref[0] = pooled * pm

    del b, ph, t


def kernel(points, W1, g1, b1, W2, g2, b2):
    B = points.shape[0]
    dense, pm, xc, yc, coords, cnt = jax.vmap(_prep)(points)

    u = W1[:, 4] - W1[:, 6]
    v = W1[:, 5] - W1[:, 7]
    w1eff = jnp.stack([W1[:, 0] + W1[:, 6], W1[:, 1] + W1[:, 7],
                       W1[:, 2], W1[:, 3]], axis=0)
    wm = jnp.concatenate([w1eff, u[None], v[None],
                          jnp.zeros((2, OUT_CH), jnp.float32)], axis=0)
    w2t = W2.T
    gb = jnp.stack([g1, b1, g2, b2], axis=0)

    grid = (B, 3, NTILES)
    feats = pl.pallas_call(
        _mlp_body,
        grid=grid,
        in_specs=[
            pl.BlockSpec(memory_space=pltpu.SMEM),
            pl.BlockSpec((1, TILE * MAX_PTS, 4), lambda b, ph, t: (b, t, 0)),
            pl.BlockSpec((1, TILE, 1), lambda b, ph, t: (b, t, 0)),
            pl.BlockSpec((1, TILE, 1), lambda b, ph, t: (b, t, 0)),
            pl.BlockSpec((1, TILE, 1), lambda b, ph, t: (b, t, 0)),
            pl.BlockSpec((8, OUT_CH), lambda b, ph, t: (0, 0)),
            pl.BlockSpec((OUT_CH, OUT_CH), lambda b, ph, t: (0, 0)),
            pl.BlockSpec((4, OUT_CH), lambda b, ph, t: (0, 0)),
        ],
        out_specs=pl.BlockSpec((1, TILE, OUT_CH), lambda b, ph, t: (b, t, 0)),
        out_shape=jax.ShapeDtypeStruct((B, MAX_PIL, OUT_CH), jnp.float32),
        scratch_shapes=[
            pltpu.VMEM((1, OUT_CH), jnp.float32),
            pltpu.VMEM((1, OUT_CH), jnp.float32),
            pltpu.VMEM((1, OUT_CH), jnp.float32),
            pltpu.VMEM((1, OUT_CH), jnp.float32),
            pltpu.VMEM((2, OUT_CH), jnp.float32),
            pltpu.VMEM((2, OUT_CH), jnp.float32),
        ],
    )(cnt, dense.reshape(B, MAX_PIL * MAX_PTS, 4),
      pm[:, :, None], xc[:, :, None], yc[:, :, None], wm, w2t, gb)
    return feats, coords


# speedup vs baseline: 5.2827x; 1.2819x over previous
import jax
import jax.numpy as jnp
import numpy as np
from jax import lax
from jax.experimental import pallas as pl
from jax.experimental.pallas import tpu as pltpu

X_MIN, Y_MIN, Z_MIN, X_MAX, Y_MAX, Z_MAX = -40.0, -40.0, -3.0, 40.0, 40.0, 1.0
PX, PY = 0.16, 0.16
NX = int(np.round((X_MAX - X_MIN) / PX))
NY = int(np.round((Y_MAX - Y_MIN) / PY))
SENT = NX * NY
MAX_PTS = 100
MAX_PIL = 12000
OUT_CH = 64
EPS = 1e-5

TILE = 96
NTILES = MAX_PIL // TILE


def _prep(p):
    n = p.shape[0]
    x, y, z, w = p[:, 0], p[:, 1], p[:, 2], p[:, 3]
    m = ((x >= X_MIN) & (x < X_MAX) & (y >= Y_MIN) & (y < Y_MAX)
         & (z >= Z_MIN) & (z < Z_MAX))
    xi = jnp.floor((x - X_MIN) / PX).astype(jnp.int32)
    yi = jnp.floor((y - Y_MIN) / PY).astype(jnp.int32)
    key = jnp.where(m, xi * NY + yi, SENT).astype(jnp.int32)
    negz = -z
    sk, snz, sx, sy, sw = lax.sort((key, negz, x, y, w), num_keys=2,
                                   is_stable=True)
    pts_s = jnp.stack([sx, sy, -snz, sw], axis=1)

    valid = sk < SENT
    newg = jnp.concatenate([jnp.ones((1,), bool), sk[1:] != sk[:-1]])
    new_valid = newg & valid
    iota = jnp.arange(n, dtype=jnp.int32)
    startkey = jnp.where(new_valid, iota, n).astype(jnp.int32)
    s_sorted, uk_sorted = lax.sort((startkey, sk), num_keys=1)

    n_valid = jnp.sum(valid.astype(jnp.int32))
    P = jnp.sum(new_valid.astype(jnp.int32))
    P_eff = jnp.minimum(P, MAX_PIL)
    s_clip = jnp.minimum(s_sorted, n_valid)
    starts = s_clip[:MAX_PIL]
    counts = s_clip[1:MAX_PIL + 1] - starts
    c100 = jnp.minimum(counts, MAX_PTS)

    pts_pad = jnp.concatenate(
        [pts_s, jnp.zeros((MAX_PTS, 4), jnp.float32)], axis=0)
    dense = lax.gather(
        pts_pad, starts[:, None],
        lax.GatherDimensionNumbers(offset_dims=(1, 2),
                                   collapsed_slice_dims=(),
                                   start_index_map=(0,)),
        slice_sizes=(MAX_PTS, 4),
        mode=lax.GatherScatterMode.CLIP)
    cm = jnp.arange(MAX_PTS, dtype=jnp.int32)[None, :] < c100[:, None]
    dense = jnp.where(cm[:, :, None], dense, 0.0)

    pm = jnp.arange(MAX_PIL, dtype=jnp.int32) < P_eff
    uk = jnp.where(pm, uk_sorted[:MAX_PIL], 0)
    ux = uk // NY
    uy = uk % NY
    xc = (ux * PX + X_MIN + PX / 2).astype(jnp.float32)
    yc = (uy * PY + Y_MIN + PY / 2).astype(jnp.float32)
    zcol = jnp.zeros((MAX_PIL,), dtype=ux.dtype)
    coords = jnp.stack([zcol, ux, uy], axis=1).astype(jnp.int64)
    cnt = jnp.maximum(P_eff * MAX_PTS, 1).astype(jnp.float32)
    return (dense, pm.astype(jnp.float32),
            xc, yc, coords, cnt)


def _mlp_body(cnt_ref, dense_ref, pm_ref, xc_ref, yc_ref, wm_ref, w2t_ref,
              gb_ref, out_ref, s1, s1q, s2, s2q, ab1, ab2):
    b = pl.program_id(0)
    ph = pl.program_id(1)
    t = pl.program_id(2)
    cnt = cnt_ref[b]

    wm = wm_ref[...]
    dense = dense_ref[0].reshape(TILE * MAX_PTS, 4)
    h1 = jnp.dot(dense, wm[0:4, :], preferred_element_type=jnp.float32)
    h1 = h1.reshape(TILE, MAX_PTS, OUT_CH)
    xc = xc_ref[0]
    yc = yc_ref[0]
    bias = xc * wm[4:5, :] + yc * wm[5:6, :]
    h1 = h1 + bias[:, None, :]
    pm = pm_ref[0]
    pm3 = pm[:, :, None]

    @pl.when(ph == 0)
    def _phase0():
        @pl.when(t == 0)
        def _z0():
            s1[...] = jnp.zeros_like(s1)
            s1q[...] = jnp.zeros_like(s1q)
        h1m = h1 * pm3
        s1[...] += jnp.sum(h1m, axis=(0, 1))[None, :]
        s1q[...] += jnp.sum(h1m * h1m, axis=(0, 1))[None, :]
        out_ref[0] = jnp.zeros((TILE, OUT_CH), jnp.float32)

    @pl.when(ph == 1)
    def _phase1():
        @pl.when(t == 0)
        def _ab1():
            mean = s1[...] / cnt
            var = s1q[...] / cnt - mean * mean
            al = gb_ref[0:1, :] * lax.rsqrt(var + EPS)
            ab1[0:1, :] = al
            ab1[1:2, :] = gb_ref[1:2, :] - mean * al
            s2[...] = jnp.zeros_like(s2)
            s2q[...] = jnp.zeros_like(s2q)
        a1 = jnp.maximum(h1 * ab1[0:1, :][None] + ab1[1:2, :][None], 0.0)
        a1m = (a1 * pm3).reshape(TILE * MAX_PTS, OUT_CH)
        h2m = jnp.dot(a1m, w2t_ref[...], preferred_element_type=jnp.float32)
        s2[...] += jnp.sum(h2m, axis=0)[None, :]
        s2q[...] += jnp.sum(h2m * h2m, axis=0)[None, :]
        out_ref[0] = jnp.zeros((TILE, OUT_CH), jnp.float32)

    @pl.when(ph == 2)
    def _phase2():
        @pl.when(t == 0)
        def _ab2():
            mean = s2[...] / cnt
            var = s2q[...] / cnt - mean * mean
            al = gb_ref[2:3, :] * lax.rsqrt(var + EPS)
            ab2[0:1, :] = al
            ab2[1:2, :] = gb_ref[3:4, :] - mean * al
        a1 = jnp.maximum(h1 * ab1[0:1, :][None] + ab1[1:2, :][None], 0.0)
        h2 = jnp.dot(a1.reshape(TILE * MAX_PTS, OUT_CH), w2t_ref[...],
                     preferred_element_type=jnp.float32)
        a2 = jnp.maximum(h2 * ab2[0:1, :] + ab2[1:2, :], 0.0)
        pooled = jnp.max(a2.reshape(TILE, MAX_PTS, OUT_CH), axis=1)
        out_ref[0] = pooled * pm

    del b, ph, t


def kernel(points, W1, g1, b1, W2, g2, b2):
    B = points.shape[0]
    dense, pm, xc, yc, coords, cnt = jax.vmap(_prep)(points)

    u = W1[:, 4] - W1[:, 6]
    v = W1[:, 5] - W1[:, 7]
    w1eff = jnp.stack([W1[:, 0] + W1[:, 6], W1[:, 1] + W1[:, 7],
                       W1[:, 2], W1[:, 3]], axis=0)
    wm = jnp.concatenate([w1eff, u[None], v[None],
                          jnp.zeros((2, OUT_CH), jnp.float32)], axis=0)
    w2t = W2.T
    gb = jnp.stack([g1, b1, g2, b2], axis=0)

    grid = (B, 3, NTILES)
    feats = pl.pallas_call(
        _mlp_body,
        grid=grid,
        in_specs=[
            pl.BlockSpec(memory_space=pltpu.SMEM),
            pl.BlockSpec((1, TILE, MAX_PTS, 4),
                         lambda b, ph, t: (b, t, 0, 0)),
            pl.BlockSpec((1, TILE, 1), lambda b, ph, t: (b, t, 0)),
            pl.BlockSpec((1, TILE, 1), lambda b, ph, t: (b, t, 0)),
            pl.BlockSpec((1, TILE, 1), lambda b, ph, t: (b, t, 0)),
            pl.BlockSpec((8, OUT_CH), lambda b, ph, t: (0, 0)),
            pl.BlockSpec((OUT_CH, OUT_CH), lambda b, ph, t: (0, 0)),
            pl.BlockSpec((4, OUT_CH), lambda b, ph, t: (0, 0)),
        ],
        out_specs=pl.BlockSpec((1, TILE, OUT_CH), lambda b, ph, t: (b, t, 0)),
        out_shape=jax.ShapeDtypeStruct((B, MAX_PIL, OUT_CH), jnp.float32),
        scratch_shapes=[
            pltpu.VMEM((1, OUT_CH), jnp.float32),
            pltpu.VMEM((1, OUT_CH), jnp.float32),
            pltpu.VMEM((1, OUT_CH), jnp.float32),
            pltpu.VMEM((1, OUT_CH), jnp.float32),
            pltpu.VMEM((2, OUT_CH), jnp.float32),
            pltpu.VMEM((2, OUT_CH), jnp.float32),
        ],
    )(cnt, dense,
      pm[:, :, None], xc[:, :, None], yc[:, :, None], wm, w2t, gb)
    return feats, coords
